# baseline (speedup 1.0000x reference)
import jax
import jax.numpy as jnp
from jax.experimental import pallas as pl
from jax.experimental.pallas import tpu as pltpu


BLOCK_N = 20000
VOCAB = 21
VOCAB_PAD = 32


def _body(x_ref, emb_ref, out_ref):
    xb = x_ref[...]
    idx = xb[:, 0].astype(jnp.int32)
    classes = jax.lax.broadcasted_iota(jnp.int32, (xb.shape[0], VOCAB_PAD), 1)
    onehot = (idx[:, None] == classes).astype(jnp.float32)
    enc = jnp.dot(onehot, emb_ref[...], preferred_element_type=jnp.float32)
    out_ref[:, :128] = enc
    out_ref[:, 128:] = xb[:, 1:]


def kernel(x, emb):
    n, f = x.shape
    hidden = emb.shape[1]
    emb_p = jnp.pad(emb, ((0, VOCAB_PAD - emb.shape[0]), (0, 0)))
    grid = (pl.cdiv(n, BLOCK_N),)
    return pl.pallas_call(
        _body,
        grid=grid,
        in_specs=[
            pl.BlockSpec((BLOCK_N, f), lambda i: (i, 0)),
            pl.BlockSpec((VOCAB_PAD, hidden), lambda i: (0, 0)),
        ],
        out_specs=pl.BlockSpec((BLOCK_N, hidden + f - 1), lambda i: (i, 0)),
        out_shape=jax.ShapeDtypeStruct((n, hidden + f - 1), jnp.float32),
        compiler_params=pltpu.CompilerParams(vmem_limit_bytes=100 * 1024 * 1024),
    )(x, emb_p)


# speedup vs baseline: 1.0022x; 1.0022x over previous
import jax
import jax.numpy as jnp
from jax.experimental import pallas as pl


BLOCK_N = 16000
VOCAB = 21
VOCAB_PAD = 32


def _body(x_ref, emb_ref, out_ref):
    xb = x_ref[...]
    idx = xb[:, 0].astype(jnp.int32)
    classes = jax.lax.broadcasted_iota(jnp.int32, (xb.shape[0], VOCAB_PAD), 1)
    onehot = (idx[:, None] == classes).astype(jnp.float32)
    enc = jnp.dot(onehot, emb_ref[...], preferred_element_type=jnp.float32)
    out_ref[:, :128] = enc
    out_ref[:, 128:] = xb[:, 1:]


def kernel(x, emb):
    n, f = x.shape
    hidden = emb.shape[1]
    emb_p = jnp.pad(emb, ((0, VOCAB_PAD - emb.shape[0]), (0, 0)))
    grid = (pl.cdiv(n, BLOCK_N),)
    return pl.pallas_call(
        _body,
        grid=grid,
        in_specs=[
            pl.BlockSpec((BLOCK_N, f), lambda i: (i, 0)),
            pl.BlockSpec((VOCAB_PAD, hidden), lambda i: (0, 0)),
        ],
        out_specs=pl.BlockSpec((BLOCK_N, hidden + f - 1), lambda i: (i, 0)),
        out_shape=jax.ShapeDtypeStruct((n, hidden + f - 1), jnp.float32),
    )(x, emb_p)
